# Initial kernel scaffold; baseline (speedup 1.0000x reference)
import jax
import jax.numpy as jnp
from jax.experimental import pallas as pl


def kernel(token_ids, step, gen_codes, gen_basis, res_weight):
    raise NotImplementedError("write your pallas kernel here")


# speedup vs baseline: 8.2662x; 8.2662x over previous
import functools

import jax
import jax.numpy as jnp
from jax import lax
from jax.experimental import pallas as pl
from jax.experimental.pallas import tpu as pltpu
from jax.experimental.pallas import tpu_sc as plsc

_VOCAB = 1000000
_D = 64
_K = 16
_RAMP_START = 1000
_RAMP_END = 10000

_NC = 2
_NS = 16
_NW = _NC * _NS


def _table_body(lam_ref, codes_ref, basis_ref, res_ref, out_ref):
    out_ref[...] = (
        jnp.dot(codes_ref[...], basis_ref[...], preferred_element_type=jnp.float32)
        + lam_ref[0] * res_ref[...]
    )


def _build_table(lam, gen_codes, gen_basis, res_weight):
    blk = 8000
    grid = _VOCAB // blk
    return pl.pallas_call(
        _table_body,
        grid=(grid,),
        in_specs=[
            pl.BlockSpec(memory_space=pltpu.SMEM),
            pl.BlockSpec((blk, _K), lambda i: (i, 0)),
            pl.BlockSpec((_K, _D), lambda i: (0, 0)),
            pl.BlockSpec((blk, _D), lambda i: (i, 0)),
        ],
        out_specs=pl.BlockSpec((blk, _D), lambda i: (i, 0)),
        out_shape=jax.ShapeDtypeStruct((_VOCAB, _D), jnp.float32),
    )(lam, gen_codes, gen_basis, res_weight)


def _make_gather(n_tokens):
    assert n_tokens % (8 * _NW) == 0
    b_per_w = n_tokens // _NW
    chunk = 512
    assert b_per_w % chunk == 0
    n_chunks = b_per_w // chunk
    mesh = plsc.VectorSubcoreMesh(core_axis_name="c", subcore_axis_name="s")

    @functools.partial(
        pl.kernel,
        mesh=mesh,
        out_type=jax.ShapeDtypeStruct((n_tokens, _D), jnp.float32),
        scratch_types=[
            pltpu.VMEM((chunk,), jnp.int32),
            pltpu.VMEM((chunk, _D), jnp.float32),
            pltpu.SemaphoreType.DMA,
        ],
        compiler_params=pltpu.CompilerParams(use_tc_tiling_on_sc=False),
    )
    def gather(idx_hbm, table_hbm, out_hbm, idx_v, rows_v, sem):
        wid = lax.axis_index("s") * _NC + lax.axis_index("c")
        base = wid * b_per_w

        def body(i, carry):
            start = base + i * chunk
            pltpu.sync_copy(idx_hbm.at[pl.ds(start, chunk)], idx_v)
            pltpu.async_copy(table_hbm.at[idx_v], rows_v, sem).wait()
            pltpu.sync_copy(rows_v, out_hbm.at[pl.ds(start, chunk)])
            return carry

        lax.fori_loop(0, n_chunks, body, 0)

    return gather


def kernel(token_ids, step, gen_codes, gen_basis, res_weight):
    b, l = token_ids.shape
    lam = jnp.clip(
        (step - _RAMP_START) / (_RAMP_END - _RAMP_START), 0.0, 1.0
    ).astype(jnp.float32).reshape(1)
    table = _build_table(lam, gen_codes, gen_basis, res_weight)
    flat_ids = token_ids.reshape(-1).astype(jnp.int32)
    out = _make_gather(flat_ids.shape[0])(flat_ids, table)
    return out.reshape(b, l, _D)


# speedup vs baseline: 25.2777x; 3.0580x over previous
import functools

import jax
import jax.numpy as jnp
from jax import lax
from jax.experimental import pallas as pl
from jax.experimental.pallas import tpu as pltpu
from jax.experimental.pallas import tpu_sc as plsc

_VOCAB = 1000000
_D = 64
_K = 16
_RAMP_START = 1000
_RAMP_END = 10000

_NC = 2
_NS = 16
_NW = _NC * _NS

_BV = 16384


def _table_body(codes_t_ref, basis_ref, res_t_ref, eye_lam_ref, out_ref):
    gen = lax.dot_general(
        codes_t_ref[...], basis_ref[...],
        (((0,), (0,)), ((), ())),
        preferred_element_type=jnp.float32,
    )
    res = lax.dot_general(
        res_t_ref[...], eye_lam_ref[...],
        (((0,), (0,)), ((), ())),
        preferred_element_type=jnp.float32,
    )
    out_ref[:, 0:_D] = gen + res


def _build_table(codes_t, basis, res_t, eye_lam):
    grid = pl.cdiv(_VOCAB, _BV)
    return pl.pallas_call(
        _table_body,
        grid=(grid,),
        in_specs=[
            pl.BlockSpec((_K, _BV), lambda i: (0, i)),
            pl.BlockSpec((_K, _D), lambda i: (0, 0)),
            pl.BlockSpec((_D, _BV), lambda i: (0, i)),
            pl.BlockSpec((_D, _D), lambda i: (0, 0)),
        ],
        out_specs=pl.BlockSpec((_BV, 2 * _D), lambda i: (i, 0)),
        out_shape=jax.ShapeDtypeStruct((_VOCAB, 2 * _D), jnp.float32),
    )(codes_t, basis, res_t, eye_lam)


def _make_gather(n_tokens):
    assert n_tokens % (8 * _NW) == 0
    b_per_w = n_tokens // _NW
    chunk = 512
    assert b_per_w % chunk == 0
    n_chunks = b_per_w // chunk
    mesh = plsc.VectorSubcoreMesh(core_axis_name="c", subcore_axis_name="s")

    half = chunk // 2
    assert n_chunks % 2 == 0

    @functools.partial(
        pl.kernel,
        mesh=mesh,
        out_type=jax.ShapeDtypeStruct((n_tokens // 2, 2 * _D), jnp.float32),
        scratch_types=[
            pltpu.VMEM((chunk,), jnp.int32),
            pltpu.VMEM((chunk,), jnp.int32),
            pltpu.VMEM((chunk, _D), jnp.float32),
            pltpu.VMEM((chunk, _D), jnp.float32),
            pltpu.SemaphoreType.DMA,
            pltpu.SemaphoreType.DMA,
        ],
        compiler_params=pltpu.CompilerParams(use_tc_tiling_on_sc=False),
    )
    def gather(idx_hbm, table_hbm, out_hbm, idx_v0, idx_v1, rows_v0, rows_v1,
               sem0, sem1):
        wid = lax.axis_index("s") * _NC + lax.axis_index("c")
        base = wid * b_per_w
        last = n_chunks - 1

        def load_idx(i, ref):
            pltpu.sync_copy(idx_hbm.at[pl.ds(base + i * chunk, chunk)], ref)

        def write_out(i, rows):
            start = base + i * chunk
            pltpu.sync_copy(
                rows.at[pl.ds(0, half)],
                out_hbm.at[pl.ds(start // 2, half), pl.ds(0, _D)],
            )
            pltpu.sync_copy(
                rows.at[pl.ds(half, half)],
                out_hbm.at[pl.ds(start // 2, half), pl.ds(_D, _D)],
            )

        load_idx(0, idx_v0)
        pltpu.async_copy(table_hbm.at[idx_v0], rows_v0, sem0)

        def body(p, carry):
            i0 = p * 2
            load_idx(i0 + 1, idx_v1)
            pltpu.async_copy(table_hbm.at[idx_v1], rows_v1, sem1)
            pltpu.make_async_copy(table_hbm.at[idx_v0], rows_v0, sem0).wait()
            write_out(i0, rows_v0)
            load_idx(jnp.minimum(i0 + 2, last), idx_v0)
            pltpu.async_copy(table_hbm.at[idx_v0], rows_v0, sem0)
            pltpu.make_async_copy(table_hbm.at[idx_v1], rows_v1, sem1).wait()
            write_out(i0 + 1, rows_v1)
            return carry

        lax.fori_loop(0, n_chunks // 2, body, 0)
        pltpu.make_async_copy(table_hbm.at[idx_v0], rows_v0, sem0).wait()

    return gather


def _retile_body(in_ref, eye_ref, out_ref):
    for j in range(8):
        t2 = lax.dot_general(
            in_ref[pl.ds(j * 256, 256), :], eye_ref[...],
            (((0,), (0,)), ((), ())),
            preferred_element_type=jnp.float32,
        )
        out_ref[0, :, pl.ds(j * 512, 256)] = t2[0:_D, :]
        out_ref[0, :, pl.ds(j * 512 + 256, 256)] = t2[_D:, :]


def _retile(out_lm2, eye256, n_b, n_l):
    return pl.pallas_call(
        _retile_body,
        grid=(n_l,),
        in_specs=[
            pl.BlockSpec((n_b // 2, 128), lambda i: (i, 0)),
            pl.BlockSpec((256, 256), lambda i: (0, 0)),
        ],
        out_specs=pl.BlockSpec((1, _D, n_b), lambda i: (i, 0, 0)),
        out_shape=jax.ShapeDtypeStruct((n_l, _D, n_b), jnp.float32),
    )(out_lm2, eye256)


def kernel(token_ids, step, gen_codes, gen_basis, res_weight):
    b, l = token_ids.shape
    lam = jnp.clip(
        (step - _RAMP_START) / (_RAMP_END - _RAMP_START), 0.0, 1.0
    ).astype(jnp.float32)
    eye_lam = lam * jnp.eye(_D, dtype=jnp.float32)
    codes_t = gen_codes.T
    res_t = res_weight.T
    table = _build_table(codes_t, gen_basis, res_t, eye_lam)
    table2 = table.reshape(2 * _VOCAB, _D)
    flat_l2 = jnp.transpose(token_ids).reshape(-1).astype(jnp.int32) * 2
    n_tok = flat_l2.shape[0]
    out_lm = _make_gather(n_tok)(flat_l2, table2)
    eye256 = jnp.eye(256, dtype=jnp.float32)
    out3 = _retile(out_lm, eye256, b, l)
    return jnp.transpose(out3, (2, 0, 1))
